# baseline (speedup 1.0000x reference)
import functools

import numpy as np
import jax
import jax.numpy as jnp
from jax import lax
from jax.experimental import pallas as pl
from jax.experimental.pallas import tpu as pltpu
from jax.experimental.pallas import tpu_sc as plsc

N_ROWS, C = 128, 512
TOTAL = N_ROWS * C
S = 100
S_PAD = 112
NCHUNK = TOTAL // 16
STREAM_LEN = 131072


def _mt_stream_np(m: int) -> np.ndarray:
    mt = np.empty(624, dtype=np.uint64)
    s = 0
    for i in range(624):
        mt[i] = s
        s = (1812433253 * (s ^ (s >> 30)) + i + 1) & 0xFFFFFFFF
    mt = mt.astype(np.uint32)
    UP, LOW, MA = np.uint32(0x80000000), np.uint32(0x7FFFFFFF), np.uint32(0x9908B0DF)

    def step(hi, lo, base):
        y = (hi & UP) | (lo & LOW)
        return base ^ (y >> np.uint32(1)) ^ ((y & np.uint32(1)) * MA)

    out = []
    for _ in range((m + 623) // 624):
        x = mt
        n1 = step(x[0:227], x[1:228], x[397:624])
        n2 = step(x[227:454], x[228:455], n1)
        n3 = step(x[454:623], x[455:624], n2[0:169])
        n4 = step(x[623:624], n1[0:1], n2[169:170])
        mt = np.concatenate([n1, n2, n3, n4])
        z = mt.copy()
        z ^= z >> np.uint32(11)
        z ^= (z << np.uint32(7)) & np.uint32(0x9D2C5680)
        z ^= (z << np.uint32(15)) & np.uint32(0xEFC60000)
        z ^= z >> np.uint32(18)
        out.append(z)
    return np.concatenate(out)[:m].view(np.int32)


_STREAM_CONST = _mt_stream_np(STREAM_LEN)

_TGT_CHUNK = 4096
_STR_CHUNK = 2048


def _g16(v, idx):
    return lax.gather(
        v, idx[:, None],
        lax.GatherDimensionNumbers(offset_dims=(), collapsed_slice_dims=(0,),
                                   start_index_map=(0,)),
        (1,), mode=lax.GatherScatterMode.PROMISE_IN_BOUNDS)


def _incl16(x):
    iota = lax.iota(jnp.int32, 16)
    s = x
    for k in (1, 2, 4, 8):
        sh = _g16(s, jnp.maximum(iota - k, 0))
        s = s + jnp.where(iota >= k, sh, 0)
    return s


def _excl16(x):
    return _incl16(x) - x


def _pm_scalar(i):
    m = i
    m = m | lax.shift_right_logical(m, 1)
    m = m | lax.shift_right_logical(m, 2)
    m = m | lax.shift_right_logical(m, 4)
    m = m | lax.shift_right_logical(m, 8)
    return m


def _sc_body(targets_ref, stream_ref, fidx_ref, cidx_ref, naux_ref,
             tbuf, sbuf, bitmask, pexcl, jarr, posmap, obuf, nbuf):
    cid_ax = lax.axis_index("c")
    sid_ax = lax.axis_index("s")

    iota = lax.iota(jnp.int32, 16)
    lane0 = iota == 0

    @pl.when(jnp.logical_and(cid_ax == 0, sid_ax == 0))
    def _work():
        def blk_body(b, _):
            pltpu.sync_copy(targets_ref.at[pl.ds(b * _TGT_CHUNK, _TGT_CHUNK)], tbuf)

            def grp_body(g, _):
                base = g * 256
                vbits = jnp.zeros((16,), jnp.int32)
                vpop = jnp.zeros((16,), jnp.int32)
                for t in range(16):
                    v = plsc.load_gather(tbuf, [iota * 16 + (base + t)])
                    flag = (v == 1).astype(jnp.int32)
                    vbits = vbits | lax.shift_left(flag, t)
                    vpop = vpop + flag
                gchunk = (b * _TGT_CHUNK // 16) + g * 16
                bitmask[pl.ds(gchunk, 16)] = vbits
                pexcl[pl.ds(gchunk, 16)] = vpop
                return 0

            lax.fori_loop(0, _TGT_CHUNK // 256, grp_body, 0)
            return 0

        lax.fori_loop(0, TOTAL // _TGT_CHUNK, blk_body, 0)

        def cs_body(k, carry):
            v = pexcl[pl.ds(k * 16, 16)]
            inc = plsc.cumsum(v)
            pexcl[pl.ds(k * 16, 16)] = inc - v + carry
            return carry + jnp.sum(v)

        n = lax.fori_loop(0, NCHUNK // 16, cs_body, jnp.int32(0))

        def scan_cond(carry):
            ck, i_s = carry
            return jnp.logical_and(i_s >= 1, ck < STREAM_LEN // _STR_CHUNK)

        def scan_body(carry):
            ck, i_s = carry
            pltpu.sync_copy(stream_ref.at[pl.ds(ck * _STR_CHUNK, _STR_CHUNK)], sbuf)

            lane15 = jnp.full((16,), 15, jnp.int32)

            def q_body(q, i_s):
                u0 = sbuf[pl.ds(q * 32, 16)]
                u1 = sbuf[pl.ds(q * 32 + 16, 16)]
                m0 = _pm_scalar(i_s)
                d0 = u0 & m0
                d1 = u1 & m0
                i0b = jnp.full((16,), i_s, jnp.int32)
                live = i0b >= 1
                a1_0 = jnp.logical_and(d0 <= i0b, live).astype(jnp.int32)
                a1_1 = jnp.logical_and(d1 <= i0b, live).astype(jnp.int32)
                s0 = _incl16(a1_0)
                s1 = _incl16(a1_1) + _g16(s0, lane15)
                iv0 = i0b - (s0 - a1_0)
                iv1 = i0b - (s1 - a1_1)
                a2_0 = jnp.logical_and(d0 <= iv0, iv0 >= 1)
                a2_1 = jnp.logical_and(d1 <= iv1, iv1 >= 1)
                a2i0 = a2_0.astype(jnp.int32)
                a2i1 = a2_1.astype(jnp.int32)
                r0 = _incl16(a2i0)
                r1 = _incl16(a2i1) + _g16(r0, lane15)
                tot2 = jnp.max(r1)
                diff = (a1_0 - a2i0) + (a1_1 - a2i1)
                i_end = i_s - tot2
                ok = jnp.logical_and(jnp.max(diff) == 0,
                                     _pm_scalar(i_end) == m0)

                def commit(_):
                    plsc.store_scatter(jarr, [i0b - (r0 - a2i0)], d0, mask=a2_0)
                    plsc.store_scatter(jarr, [i0b - (r1 - a2i1)], d1, mask=a2_1)
                    return i_end

                def fallback(_):
                    i_vec = i0b
                    m = jnp.full((16,), m0, jnp.int32)
                    for t in range(32):
                        uu = u0 if t < 16 else u1
                        dt = _g16(uu, jnp.full((16,), t % 16, jnp.int32)) & m
                        acc = jnp.logical_and(dt <= i_vec, i_vec >= 1)
                        plsc.store_scatter(jarr, [i_vec], dt,
                                           mask=jnp.logical_and(acc, lane0))
                        i_vec = i_vec - acc.astype(jnp.int32)
                        mh = lax.shift_right_logical(m, 1)
                        m = jnp.where(i_vec <= mh, mh, m)
                    return jnp.max(i_vec)

                return lax.cond(ok, commit, fallback, 0)

            i_s = lax.fori_loop(0, _STR_CHUNK // 32, q_body, i_s)
            return (ck + 1, i_s)

        lax.while_loop(scan_cond, scan_body, (jnp.int32(0), n - 1))

        def pz_body(k, _):
            posmap[pl.ds(k * 16, 16)] = jnp.zeros((16,), jnp.int32)
            return 0

        lax.fori_loop(0, NCHUNK // 16, pz_body, 0)
        posmap[pl.ds(0, 16)] = jnp.where(iota < S_PAD // 16, 0xFFFF, 0)

        pos = tuple(iota + 16 * v for v in range(7))
        nm1 = jnp.full((16,), n - 1, jnp.int32)
        one16 = jnp.ones((16,), jnp.int32)

        def trk_body(ci, pos):
            jv = jarr[pl.ds(ci * 16, 16)] & 0xFFFF
            jw = plsc.load_gather(posmap, [lax.shift_right_logical(jv, 4)])
            jbit = lax.shift_right_logical(jw, jv & 15) & 1
            iw = plsc.load_gather(posmap, [jnp.full((16,), ci, jnp.int32)])
            hit = jnp.sum(jbit | iw) > 0

            def do_serial(pos):
                before = pos
                pos = list(pos)
                for t in range(16):
                    ib = jnp.full((16,), ci * 16 + t, jnp.int32)
                    jb = _g16(jv, jnp.full((16,), t, jnp.int32))
                    ok = jnp.logical_and(ib >= 1, ib <= nm1)
                    for v in range(7):
                        pv = pos[v]
                        pv = jnp.where(jnp.logical_and(ok, pv == ib), jb,
                                       jnp.where(jnp.logical_and(ok, pv == jb), ib, pv))
                        pos[v] = pv
                for v in range(7):
                    bv, av = before[v], pos[v]
                    ch = bv != av
                    plsc.addupdate_scatter(
                        posmap, [lax.shift_right_logical(bv, 4)],
                        -lax.shift_left(one16, bv & 15), mask=ch)
                    plsc.addupdate_scatter(
                        posmap, [lax.shift_right_logical(av, 4)],
                        lax.shift_left(one16, av & 15), mask=ch)
                return tuple(pos)

            return lax.cond(hit, do_serial, lambda p: p, pos)

        nchunks = lax.div(n + 15, jnp.int32(16))
        pos = lax.fori_loop(0, nchunks, trk_body, pos)

        use_perm = n > S
        for v in range(7):
            slot = iota + 16 * v
            sel = jnp.where(use_perm, pos[v], slot)
            lo = jnp.zeros((16,), jnp.int32)
            step = NCHUNK // 2
            while step >= 1:
                cand = lo + step
                candc = jnp.minimum(cand, NCHUNK - 1)
                val = plsc.load_gather(pexcl, [candc])
                take = jnp.logical_and(cand <= NCHUNK - 1, val <= sel)
                lo = jnp.where(take, cand, lo)
                step //= 2
            base_rank = plsc.load_gather(pexcl, [lo])
            r_in = sel - base_rank
            w = plsc.load_gather(bitmask, [lo])
            cnt = jnp.zeros((16,), jnp.int32)
            pos_in = jnp.zeros((16,), jnp.int32)
            found = jnp.zeros((16,), jnp.bool_)
            for t in range(16):
                bit = lax.shift_right_logical(w, t) & 1
                cnt = cnt + bit
                hit = jnp.logical_and(jnp.logical_not(found),
                                      jnp.logical_and(bit == 1, cnt == r_in + 1))
                pos_in = jnp.where(hit, t, pos_in)
                found = jnp.logical_or(found, hit)
            flat = jnp.where(found, lo * 16 + pos_in, TOTAL - 1)
            obuf[pl.ds(16 * v, 16)] = flat
            obuf[pl.ds(S_PAD + 16 * v, 16)] = flat & (C - 1)
        nbuf[...] = jnp.full((16,), n, jnp.int32)

        pltpu.sync_copy(obuf.at[pl.ds(0, S_PAD)], fidx_ref)
        pltpu.sync_copy(obuf.at[pl.ds(S_PAD, S_PAD)], cidx_ref)
        pltpu.sync_copy(nbuf, naux_ref)


_sc_sample = pl.kernel(
    _sc_body,
    out_type=(
        jax.ShapeDtypeStruct((S_PAD,), jnp.int32),
        jax.ShapeDtypeStruct((S_PAD,), jnp.int32),
        jax.ShapeDtypeStruct((16,), jnp.int32),
    ),
    mesh=plsc.VectorSubcoreMesh(core_axis_name="c", subcore_axis_name="s"),
    scratch_types=(
        pltpu.VMEM((_TGT_CHUNK,), jnp.int32),
        pltpu.VMEM((_STR_CHUNK,), jnp.int32),
        pltpu.VMEM((NCHUNK,), jnp.int32),
        pltpu.VMEM((NCHUNK,), jnp.int32),
        pltpu.VMEM((TOTAL,), jnp.int32),
        pltpu.VMEM((NCHUNK,), jnp.int32),
        pltpu.VMEM((2 * S_PAD,), jnp.int32),
        pltpu.VMEM((16,), jnp.int32),
    ),
    compiler_params=pltpu.CompilerParams(needs_layout_passes=False),
    name="sc_mt_sample",
)


def _tc_body(fidx_ref, cidx_ref, naux_ref, scores_blk, ref_blk, num_acc, cnt_acc):
    s = pl.program_id(0)

    @pl.when(s == 0)
    def _init():
        num_acc[...] = jnp.zeros_like(num_acc)
        cnt_acc[...] = jnp.zeros_like(cnt_acc)

    l_row = scores_blk[0]
    r_row = ref_blk[0]

    eye = (lax.broadcasted_iota(jnp.int32, (C, C), 0)
           == lax.broadcasted_iota(jnp.int32, (C, C), 1)).astype(jnp.float32)
    l_col = lax.dot_general(eye, l_row, (((1,), (1,)), ((), ())),
                            preferred_element_type=jnp.float32)

    rel = jnp.broadcast_to(l_row, (C, C)) - jnp.broadcast_to(l_col, (C, C))
    msk = jnp.logical_and(jnp.broadcast_to(r_row, (C, C)) > 0, rel < 0)

    valid = s < naux_ref[0]
    gate_f = jnp.where(valid, 1.0, 0.0)
    gate_i = jnp.where(valid, 1, 0)
    num_acc[...] += jnp.sum(jnp.where(msk, rel, 0.0), axis=0, keepdims=True) * gate_f
    cnt_acc[...] += jnp.sum(msk.astype(jnp.int32), axis=0, keepdims=True) * gate_i


_tc_loss = pl.pallas_call(
    _tc_body,
    grid_spec=pltpu.PrefetchScalarGridSpec(
        num_scalar_prefetch=3,
        grid=(S,),
        in_specs=[
            pl.BlockSpec((1, 1, C), lambda s, fidx, cidx, naux: (fidx[s], 0, 0)),
            pl.BlockSpec((1, 1, C), lambda s, fidx, cidx, naux: (cidx[s], 0, 0)),
        ],
        out_specs=[
            pl.BlockSpec((1, C), lambda s, fidx, cidx, naux: (0, 0)),
            pl.BlockSpec((1, C), lambda s, fidx, cidx, naux: (0, 0)),
        ],
    ),
    out_shape=(
        jax.ShapeDtypeStruct((1, C), jnp.float32),
        jax.ShapeDtypeStruct((1, C), jnp.int32),
    ),
    name="tc_margin_loss",
)


def kernel(scores, targets, ref):
    targets_i32 = targets.reshape(TOTAL).astype(jnp.int32)
    stream = jnp.asarray(_STREAM_CONST)

    fidx, cidx, naux = _sc_sample(targets_i32, stream)

    scores2 = scores.reshape(TOTAL, 1, C)
    ref2 = ref.reshape(C, 1, C)
    num_v, cnt_v = _tc_loss(fidx, cidx, naux, scores2, ref2)

    num = jnp.sum(num_v)
    cnt = jnp.sum(cnt_v).astype(jnp.float32)
    return jnp.where(cnt > 0, -num / jnp.maximum(cnt, 1.0), jnp.float32(0.0))


# speedup vs baseline: 1.0047x; 1.0047x over previous
import functools

import numpy as np
import jax
import jax.numpy as jnp
from jax import lax
from jax.experimental import pallas as pl
from jax.experimental.pallas import tpu as pltpu
from jax.experimental.pallas import tpu_sc as plsc

N_ROWS, C = 128, 512
TOTAL = N_ROWS * C
S = 100
S_PAD = 112
NCHUNK = TOTAL // 16
STREAM_LEN = 131072


def _mt_stream_np(m: int) -> np.ndarray:
    mt = np.empty(624, dtype=np.uint64)
    s = 0
    for i in range(624):
        mt[i] = s
        s = (1812433253 * (s ^ (s >> 30)) + i + 1) & 0xFFFFFFFF
    mt = mt.astype(np.uint32)
    UP, LOW, MA = np.uint32(0x80000000), np.uint32(0x7FFFFFFF), np.uint32(0x9908B0DF)

    def step(hi, lo, base):
        y = (hi & UP) | (lo & LOW)
        return base ^ (y >> np.uint32(1)) ^ ((y & np.uint32(1)) * MA)

    out = []
    for _ in range((m + 623) // 624):
        x = mt
        n1 = step(x[0:227], x[1:228], x[397:624])
        n2 = step(x[227:454], x[228:455], n1)
        n3 = step(x[454:623], x[455:624], n2[0:169])
        n4 = step(x[623:624], n1[0:1], n2[169:170])
        mt = np.concatenate([n1, n2, n3, n4])
        z = mt.copy()
        z ^= z >> np.uint32(11)
        z ^= (z << np.uint32(7)) & np.uint32(0x9D2C5680)
        z ^= (z << np.uint32(15)) & np.uint32(0xEFC60000)
        z ^= z >> np.uint32(18)
        out.append(z)
    return np.concatenate(out)[:m].view(np.int32)


_STREAM_CONST = _mt_stream_np(STREAM_LEN)

_TGT_CHUNK = 32768
_STR_CHUNK = 2048
_N_STR_CHUNKS = STREAM_LEN // _STR_CHUNK


def _g16(v, idx):
    return lax.gather(
        v, idx[:, None],
        lax.GatherDimensionNumbers(offset_dims=(), collapsed_slice_dims=(0,),
                                   start_index_map=(0,)),
        (1,), mode=lax.GatherScatterMode.PROMISE_IN_BOUNDS)


def _incl16(x):
    iota = lax.iota(jnp.int32, 16)
    s = x
    for k in (1, 2, 4, 8):
        sh = _g16(s, jnp.maximum(iota - k, 0))
        s = s + jnp.where(iota >= k, sh, 0)
    return s


def _excl16(x):
    return _incl16(x) - x


def _pm_scalar(i):
    m = i
    m = m | lax.shift_right_logical(m, 1)
    m = m | lax.shift_right_logical(m, 2)
    m = m | lax.shift_right_logical(m, 4)
    m = m | lax.shift_right_logical(m, 8)
    return m


def _sc_body(targets_ref, stream_ref, fidx_ref, cidx_ref, naux_ref,
             tbuf, sbuf, bitmask, pexcl, jarr, posmap, obuf, nbuf, sem):
    cid_ax = lax.axis_index("c")
    sid_ax = lax.axis_index("s")

    iota = lax.iota(jnp.int32, 16)
    lane0 = iota == 0

    @pl.when(jnp.logical_and(cid_ax == 0, sid_ax == 0))
    def _work():
        def blk_body(b, _):
            pltpu.sync_copy(targets_ref.at[pl.ds(b * _TGT_CHUNK, _TGT_CHUNK)], tbuf)

            def grp_body(g, _):
                base = g * 256
                vbits = jnp.zeros((16,), jnp.int32)
                vpop = jnp.zeros((16,), jnp.int32)
                for t in range(16):
                    v = plsc.load_gather(tbuf, [iota * 16 + (base + t)])
                    flag = (v == 1).astype(jnp.int32)
                    vbits = vbits | lax.shift_left(flag, t)
                    vpop = vpop + flag
                gchunk = (b * _TGT_CHUNK // 16) + g * 16
                bitmask[pl.ds(gchunk, 16)] = vbits
                pexcl[pl.ds(gchunk, 16)] = vpop
                return 0

            lax.fori_loop(0, _TGT_CHUNK // 256, grp_body, 0)
            return 0

        lax.fori_loop(0, TOTAL // _TGT_CHUNK, blk_body, 0)

        def cs_body(k, carry):
            v = pexcl[pl.ds(k * 16, 16)]
            inc = plsc.cumsum(v)
            pexcl[pl.ds(k * 16, 16)] = inc - v + carry
            return carry + jnp.sum(v)

        n = lax.fori_loop(0, NCHUNK // 16, cs_body, jnp.int32(0))

        def _str_copy(ck):
            return pltpu.make_async_copy(
                stream_ref.at[pl.ds(ck * _STR_CHUNK, _STR_CHUNK)],
                sbuf.at[pl.ds((ck % 2) * _STR_CHUNK, _STR_CHUNK)], sem)

        def scan_cond(carry):
            ck, i_s = carry
            return jnp.logical_and(i_s >= 1, ck < _N_STR_CHUNKS)

        def scan_body(carry):
            ck, i_s = carry
            _str_copy(ck).wait()

            @pl.when(ck + 1 < _N_STR_CHUNKS)
            def _prefetch():
                _str_copy(ck + 1).start()

            base = (ck % 2) * _STR_CHUNK
            lane15 = jnp.full((16,), 15, jnp.int32)

            def q_body(q, i_s):
                u0 = sbuf[pl.ds(base + q * 32, 16)]
                u1 = sbuf[pl.ds(base + q * 32 + 16, 16)]
                m0 = _pm_scalar(i_s)
                d0 = u0 & m0
                d1 = u1 & m0
                i0b = jnp.full((16,), i_s, jnp.int32)
                live = i0b >= 1
                a1_0 = jnp.logical_and(d0 <= i0b, live).astype(jnp.int32)
                a1_1 = jnp.logical_and(d1 <= i0b, live).astype(jnp.int32)
                s0 = _incl16(a1_0)
                s1 = _incl16(a1_1) + _g16(s0, lane15)
                iv0 = i0b - (s0 - a1_0)
                iv1 = i0b - (s1 - a1_1)
                a2_0 = jnp.logical_and(d0 <= iv0, iv0 >= 1)
                a2_1 = jnp.logical_and(d1 <= iv1, iv1 >= 1)
                a2i0 = a2_0.astype(jnp.int32)
                a2i1 = a2_1.astype(jnp.int32)
                r0 = _incl16(a2i0)
                r1 = _incl16(a2i1) + _g16(r0, lane15)
                diff = (a1_0 - a2i0) + (a1_1 - a2i1)
                zm = jnp.max(diff * 131072 + r1)
                tot2 = zm & 131071
                i_end = i_s - tot2
                ok = jnp.logical_and(zm < 131072,
                                     _pm_scalar(i_end) == m0)

                def commit(_):
                    plsc.store_scatter(jarr, [i0b - (r0 - a2i0)], d0, mask=a2_0)
                    plsc.store_scatter(jarr, [i0b - (r1 - a2i1)], d1, mask=a2_1)
                    return i_end

                def fallback(_):
                    i_vec = i0b
                    m = jnp.full((16,), m0, jnp.int32)
                    for t in range(32):
                        uu = u0 if t < 16 else u1
                        dt = _g16(uu, jnp.full((16,), t % 16, jnp.int32)) & m
                        acc = jnp.logical_and(dt <= i_vec, i_vec >= 1)
                        plsc.store_scatter(jarr, [i_vec], dt,
                                           mask=jnp.logical_and(acc, lane0))
                        i_vec = i_vec - acc.astype(jnp.int32)
                        mh = lax.shift_right_logical(m, 1)
                        m = jnp.where(i_vec <= mh, mh, m)
                    return jnp.max(i_vec)

                return lax.cond(ok, commit, fallback, 0)

            i_s = lax.fori_loop(0, _STR_CHUNK // 32, q_body, i_s)
            return (ck + 1, i_s)

        _str_copy(jnp.int32(0)).start()
        ck_exit, _ = lax.while_loop(scan_cond, scan_body, (jnp.int32(0), n - 1))

        @pl.when(ck_exit < _N_STR_CHUNKS)
        def _drain():
            _str_copy(ck_exit).wait()

        def pz_body(k, _):
            posmap[pl.ds(k * 16, 16)] = jnp.zeros((16,), jnp.int32)
            return 0

        lax.fori_loop(0, NCHUNK // 16, pz_body, 0)
        posmap[pl.ds(0, 16)] = jnp.where(iota < S_PAD // 16, 0xFFFF, 0)

        pos = tuple(iota + 16 * v for v in range(7))
        nm1 = jnp.full((16,), n - 1, jnp.int32)
        one16 = jnp.ones((16,), jnp.int32)

        def trk_body(ci, pos):
            jv = jarr[pl.ds(ci * 16, 16)] & 0xFFFF
            jw = plsc.load_gather(posmap, [lax.shift_right_logical(jv, 4)])
            jbit = lax.shift_right_logical(jw, jv & 15) & 1
            iw = plsc.load_gather(posmap, [jnp.full((16,), ci, jnp.int32)])
            hit = jnp.sum(jbit | iw) > 0

            def do_serial(pos):
                before = pos
                pos = list(pos)
                for t in range(16):
                    ib = jnp.full((16,), ci * 16 + t, jnp.int32)
                    jb = _g16(jv, jnp.full((16,), t, jnp.int32))
                    ok = jnp.logical_and(ib >= 1, ib <= nm1)
                    for v in range(7):
                        pv = pos[v]
                        pv = jnp.where(jnp.logical_and(ok, pv == ib), jb,
                                       jnp.where(jnp.logical_and(ok, pv == jb), ib, pv))
                        pos[v] = pv
                for v in range(7):
                    bv, av = before[v], pos[v]
                    ch = bv != av
                    plsc.addupdate_scatter(
                        posmap, [lax.shift_right_logical(bv, 4)],
                        -lax.shift_left(one16, bv & 15), mask=ch)
                    plsc.addupdate_scatter(
                        posmap, [lax.shift_right_logical(av, 4)],
                        lax.shift_left(one16, av & 15), mask=ch)
                return tuple(pos)

            return lax.cond(hit, do_serial, lambda p: p, pos)

        nchunks = lax.div(n + 15, jnp.int32(16))
        pos = lax.fori_loop(0, nchunks, trk_body, pos)

        use_perm = n > S
        for v in range(7):
            slot = iota + 16 * v
            sel = jnp.where(use_perm, pos[v], slot)
            lo = jnp.zeros((16,), jnp.int32)
            step = NCHUNK // 2
            while step >= 1:
                cand = lo + step
                candc = jnp.minimum(cand, NCHUNK - 1)
                val = plsc.load_gather(pexcl, [candc])
                take = jnp.logical_and(cand <= NCHUNK - 1, val <= sel)
                lo = jnp.where(take, cand, lo)
                step //= 2
            base_rank = plsc.load_gather(pexcl, [lo])
            r_in = sel - base_rank
            w = plsc.load_gather(bitmask, [lo])
            cnt = jnp.zeros((16,), jnp.int32)
            pos_in = jnp.zeros((16,), jnp.int32)
            found = jnp.zeros((16,), jnp.bool_)
            for t in range(16):
                bit = lax.shift_right_logical(w, t) & 1
                cnt = cnt + bit
                hit = jnp.logical_and(jnp.logical_not(found),
                                      jnp.logical_and(bit == 1, cnt == r_in + 1))
                pos_in = jnp.where(hit, t, pos_in)
                found = jnp.logical_or(found, hit)
            flat = jnp.where(found, lo * 16 + pos_in, TOTAL - 1)
            obuf[pl.ds(16 * v, 16)] = flat
            obuf[pl.ds(S_PAD + 16 * v, 16)] = flat & (C - 1)
        nbuf[...] = jnp.full((16,), n, jnp.int32)

        pltpu.sync_copy(obuf.at[pl.ds(0, S_PAD)], fidx_ref)
        pltpu.sync_copy(obuf.at[pl.ds(S_PAD, S_PAD)], cidx_ref)
        pltpu.sync_copy(nbuf, naux_ref)


_sc_sample = pl.kernel(
    _sc_body,
    out_type=(
        jax.ShapeDtypeStruct((S_PAD,), jnp.int32),
        jax.ShapeDtypeStruct((S_PAD,), jnp.int32),
        jax.ShapeDtypeStruct((16,), jnp.int32),
    ),
    mesh=plsc.VectorSubcoreMesh(core_axis_name="c", subcore_axis_name="s"),
    scratch_types=(
        pltpu.VMEM((_TGT_CHUNK,), jnp.int32),
        pltpu.VMEM((2 * _STR_CHUNK,), jnp.int32),
        pltpu.VMEM((NCHUNK,), jnp.int32),
        pltpu.VMEM((NCHUNK,), jnp.int32),
        pltpu.VMEM((TOTAL,), jnp.int32),
        pltpu.VMEM((NCHUNK,), jnp.int32),
        pltpu.VMEM((2 * S_PAD,), jnp.int32),
        pltpu.VMEM((16,), jnp.int32),
        pltpu.SemaphoreType.DMA,
    ),
    compiler_params=pltpu.CompilerParams(needs_layout_passes=False),
    name="sc_mt_sample",
)


def _tc_body(fidx_ref, cidx_ref, naux_ref, scores_blk, ref_blk, num_acc, cnt_acc):
    s = pl.program_id(0)

    @pl.when(s == 0)
    def _init():
        num_acc[...] = jnp.zeros_like(num_acc)
        cnt_acc[...] = jnp.zeros_like(cnt_acc)

    l_row = scores_blk[0]
    r_row = ref_blk[0]

    eye = (lax.broadcasted_iota(jnp.int32, (C, C), 0)
           == lax.broadcasted_iota(jnp.int32, (C, C), 1)).astype(jnp.float32)
    l_col = lax.dot_general(eye, l_row, (((1,), (1,)), ((), ())),
                            preferred_element_type=jnp.float32)

    rel = jnp.broadcast_to(l_row, (C, C)) - jnp.broadcast_to(l_col, (C, C))
    msk = jnp.logical_and(jnp.broadcast_to(r_row, (C, C)) > 0, rel < 0)

    valid = s < naux_ref[0]
    gate_f = jnp.where(valid, 1.0, 0.0)
    gate_i = jnp.where(valid, 1, 0)
    num_acc[...] += jnp.sum(jnp.where(msk, rel, 0.0), axis=0, keepdims=True) * gate_f
    cnt_acc[...] += jnp.sum(msk.astype(jnp.int32), axis=0, keepdims=True) * gate_i


_tc_loss = pl.pallas_call(
    _tc_body,
    grid_spec=pltpu.PrefetchScalarGridSpec(
        num_scalar_prefetch=3,
        grid=(S,),
        in_specs=[
            pl.BlockSpec((1, 1, C), lambda s, fidx, cidx, naux: (fidx[s], 0, 0)),
            pl.BlockSpec((1, 1, C), lambda s, fidx, cidx, naux: (cidx[s], 0, 0)),
        ],
        out_specs=[
            pl.BlockSpec((1, C), lambda s, fidx, cidx, naux: (0, 0)),
            pl.BlockSpec((1, C), lambda s, fidx, cidx, naux: (0, 0)),
        ],
    ),
    out_shape=(
        jax.ShapeDtypeStruct((1, C), jnp.float32),
        jax.ShapeDtypeStruct((1, C), jnp.int32),
    ),
    name="tc_margin_loss",
)


def kernel(scores, targets, ref):
    targets_i32 = targets.reshape(TOTAL).astype(jnp.int32)
    stream = jnp.asarray(_STREAM_CONST)

    fidx, cidx, naux = _sc_sample(targets_i32, stream)

    scores2 = scores.reshape(TOTAL, 1, C)
    ref2 = ref.reshape(C, 1, C)
    num_v, cnt_v = _tc_loss(fidx, cidx, naux, scores2, ref2)

    num = jnp.sum(num_v)
    cnt = jnp.sum(cnt_v).astype(jnp.float32)
    return jnp.where(cnt > 0, -num / jnp.maximum(cnt, 1.0), jnp.float32(0.0))
